# Initial kernel scaffold; baseline (speedup 1.0000x reference)
import jax
import jax.numpy as jnp
from jax.experimental import pallas as pl


def kernel(user_emb_glo, item_emb_glo, user_emb_loc, item_emb_loc, edges_aux1, edges_aux2, edges_tar, edges_all, edges_aug, batch_data):
    raise NotImplementedError("write your pallas kernel here")


# speedup vs baseline: 1.4054x; 1.4054x over previous
import jax
import jax.numpy as jnp
from jax import lax
from jax.experimental import pallas as pl
from jax.experimental.pallas import tpu as pltpu

_N_USERS = 25000
_N_ITEMS = 25000
_D = 64
_LAYERS = 2
_TEMP_S = 0.2
_CON_S = 0.1
_N_NODES = (_N_USERS + 1) + (_N_ITEMS + 1)
_NB = 1024


def _lightgcn(emb, edges, layers):
    u = edges[0]
    i = edges[1] + (_N_USERS + 1)
    s = jnp.concatenate([u, i])
    d = jnp.concatenate([i, u])
    deg = jnp.zeros((_N_NODES,), emb.dtype).at[s].add(1.0)
    dinv = jnp.where(deg > 0, 1.0 / jnp.sqrt(jnp.maximum(deg, 1.0)), 0.0)
    norm = dinv[s] * dinv[d]
    acc = emb
    x = emb
    for _ in range(layers):
        x = jnp.zeros_like(x).at[d].add(x[s] * norm[:, None])
        acc = acc + x
    return acc / (layers + 1.0)


def _loss_body(pu_ref, au_ref, pi_ref, ai_ref, out_ref):
    def cl(p, a):
        pn = p / jnp.maximum(jnp.sqrt(jnp.sum(p * p, axis=1, keepdims=True)), 1e-12)
        an = a / jnp.maximum(jnp.sqrt(jnp.sum(a * a, axis=1, keepdims=True)), 1e-12)
        pos = jnp.exp(jnp.sum(pn * an, axis=1) / _TEMP_S)
        scores = lax.dot_general(pn, an, (((1,), (1,)), ((), ())),
                                 preferred_element_type=jnp.float32)
        ttl = jnp.sum(jnp.exp(scores / _TEMP_S), axis=1)
        return -jnp.mean(jnp.log(pos / ttl))

    c = (cl(pu_ref[...], au_ref[...]) + cl(pi_ref[...], ai_ref[...])) / 2.0
    out_ref[0, 0] = _CON_S * c


def _con_loss_pallas(pu, au, pi, ai):
    out = pl.pallas_call(
        _loss_body,
        out_shape=jax.ShapeDtypeStruct((1, 1), jnp.float32),
        out_specs=pl.BlockSpec(memory_space=pltpu.SMEM),
    )(pu, au, pi, ai)
    return out[0, 0]


def kernel(user_emb_glo, item_emb_glo, user_emb_loc, item_emb_loc,
           edges_aux1, edges_aux2, edges_tar, edges_all, edges_aug, batch_data):
    del user_emb_glo, item_emb_glo, edges_all, batch_data
    emb_loc = jnp.concatenate([user_emb_loc, item_emb_loc], axis=0)
    beh = [_lightgcn(emb_loc, e, _LAYERS) for e in (edges_aux1, edges_aux2, edges_tar)]
    u_beh = [b[: _N_USERS + 1] for b in beh]
    i_beh = [b[_N_USERS + 1:] for b in beh]
    user_loc = jnp.mean(jnp.stack(u_beh, axis=0), axis=0)
    item_loc = jnp.mean(jnp.stack(i_beh, axis=0), axis=0)
    aug = _lightgcn(emb_loc, edges_aug, _LAYERS)
    u_aug = aug[: _N_USERS + 1]
    i_aug = aug[_N_USERS + 1:]

    idx_u = jax.random.permutation(jax.random.key(1), _N_USERS + 1)[:_NB]
    idx_i = jax.random.permutation(jax.random.key(2), _N_USERS + 1)[:_NB]
    loss = _con_loss_pallas(user_loc[idx_u], u_aug[idx_u],
                            item_loc[idx_i], i_aug[idx_i])
    return loss


# speedup vs baseline: 18.9466x; 13.4810x over previous
import functools

import jax
import jax.numpy as jnp
from jax import lax
from jax.experimental import pallas as pl
from jax.experimental.pallas import tpu as pltpu
from jax.experimental.pallas import tpu_sc as plsc

_N_USERS = 25000
_D = 64
_TEMP_S = 0.2
_CON_S = 0.1
_NB = 1024

_N_REAL = 2 * (_N_USERS + 1)
_N_PAD = 50176
_HALF = _N_PAD // 2
_TRASH = 384
_ACC_ROWS = _HALF + _TRASH
_PAD_NODE = _N_REAL
_CH = 8


def _prep_edges(edges, rpad, g):
    u = edges[0].astype(jnp.int32)
    i = edges[1].astype(jnp.int32) + (_N_USERS + 1)
    s = jnp.concatenate([u, i])
    d = jnp.concatenate([i, u])
    pad = rpad * 128 - s.shape[0]
    s_deg = jnp.concatenate([s, jnp.full((pad,), _PAD_NODE, jnp.int32)])
    s_gat = s_deg + g * _N_PAD
    d_sc = jnp.concatenate([d, jnp.full((pad,), _N_PAD + 8, jnp.int32)])
    return (s_deg.reshape(rpad, 128), s_gat.reshape(rpad, 128),
            d_sc.reshape(rpad, 128))


def _deg_body(s0, s1, s2, s3, out, degs, sbuf, ones, zbuf):
    c = lax.axis_index("c")
    sid = lax.axis_index("s")
    for k in range(8):
        ones[pl.ds(k * 16, 16)] = jnp.ones((16,), jnp.float32)
    for k in range(64):
        zbuf[pl.ds(k * 16, 16)] = jnp.zeros((16,), jnp.float32)
    for g, sref in enumerate((s0, s1, s2, s3)):
        base = sid * (_N_PAD // 16)
        pltpu.sync_copy(zbuf, degs.at[pl.ds(base, 1024)])
        pltpu.sync_copy(zbuf, degs.at[pl.ds(base + 1024, 1024)])
        pltpu.sync_copy(zbuf, degs.at[pl.ds(base + 2048, 1024)])
        pltpu.sync_copy(zbuf.at[pl.ds(0, 64)], degs.at[pl.ds(base + 3072, 64)])
        plsc.subcore_barrier()
        rows = sref.shape[0]
        per_w = rows // 32
        r0 = (c * 16 + sid) * per_w

        def chunk(cc, carry):
            j0 = r0 + cc * _CH
            pltpu.sync_copy(sref.at[pl.ds(j0, _CH)], sbuf)
            for j in range(_CH):
                pltpu.sync_copy(ones, degs.at[sbuf.at[j]], add=True)
            return carry

        lax.fori_loop(0, per_w // _CH, chunk, 0)
        plsc.subcore_barrier()
        p = (c * 4 + g) * _N_PAD
        pltpu.sync_copy(degs.at[pl.ds(sid * 3072, 3072)],
                        out.at[pl.ds(p + sid * 3072, 3072)])

        @pl.when(sid == 15)
        def _tail():
            pltpu.sync_copy(degs.at[pl.ds(49152, 1024)],
                            out.at[pl.ds(p + 49152, 1024)])
        plsc.subcore_barrier()


def _deg_call(s_degs):
    mesh = plsc.VectorSubcoreMesh(core_axis_name="c", subcore_axis_name="s")
    f = pl.kernel(
        _deg_body,
        out_type=jax.ShapeDtypeStruct((8 * _N_PAD,), jnp.float32),
        mesh=mesh,
        scratch_types=[
            pltpu.VMEM_SHARED((_N_PAD,), jnp.float32),
            pltpu.VMEM((_CH, 128), jnp.int32),
            pltpu.VMEM((128,), jnp.float32),
            pltpu.VMEM((1024,), jnp.float32),
        ],
    )
    return f(*s_degs)


def _prop_body(yflat, s0, d0, s1, d1, s2, d2, s3, d3, out,
               accs, sbuf, dbuf, dlbuf, rows_a, rows_b, zrows, sem_a, sem_b):
    c = lax.axis_index("c")
    sid = lax.axis_index("s")
    lo = c * _HALF
    lane = lax.iota(jnp.int32, 16)
    trash = _HALF + sid * 16
    for r in range(16):
        for k in range(4):
            zrows[r, pl.ds(k * 16, 16)] = jnp.zeros((16,), jnp.float32)
    my_rows = _ACC_ROWS // 16

    for g, (sref, dref) in enumerate(((s0, d0), (s1, d1), (s2, d2), (s3, d3))):
        def zero(z, carry):
            pltpu.sync_copy(zrows, accs.at[pl.ds(sid * my_rows + z * 16, 16)])
            return carry
        lax.fori_loop(0, my_rows // 16, zero, 0)
        plsc.subcore_barrier()

        rows = sref.shape[0]
        per_w = rows // 16
        r0 = sid * per_w

        def chunk(cc, carry):
            j0 = r0 + cc * _CH
            pltpu.sync_copy(sref.at[pl.ds(j0, _CH)], sbuf)
            pltpu.sync_copy(dref.at[pl.ds(j0, _CH)], dbuf)

            def remap(k, cy):
                j = k // 8
                m = k - j * 8
                dv = dbuf[j, pl.ds(m * 16, 16)] - lo
                ok = (dv >= 0) & (dv < _HALF)
                dlbuf[j, pl.ds(m * 16, 16)] = jnp.where(ok, dv, trash + lane)
                return cy

            lax.fori_loop(0, _CH * 8, remap, 0)

            def pair(t, cy):
                ja = 2 * t
                jb = 2 * t + 1
                da = pltpu.async_copy(yflat.at[sbuf.at[ja]], rows_a, sem_a)
                db = pltpu.async_copy(yflat.at[sbuf.at[jb]], rows_b, sem_b)
                da.wait()
                pltpu.sync_copy(rows_a, accs.at[dlbuf.at[ja]], add=True)
                db.wait()
                pltpu.sync_copy(rows_b, accs.at[dlbuf.at[jb]], add=True)
                return cy

            lax.fori_loop(0, _CH // 2, pair, 0)
            return carry

        lax.fori_loop(0, per_w // _CH, chunk, 0)
        plsc.subcore_barrier()

        off = sid * (_HALF // 16)
        pltpu.sync_copy(accs.at[pl.ds(off, _HALF // 16)],
                        out.at[g, pl.ds(lo + off, _HALF // 16), :])
        plsc.subcore_barrier()


def _prop_call(yflat, edge_refs):
    mesh = plsc.VectorSubcoreMesh(core_axis_name="c", subcore_axis_name="s")
    f = pl.kernel(
        _prop_body,
        out_type=jax.ShapeDtypeStruct((4, _N_PAD, _D), jnp.float32),
        mesh=mesh,
        scratch_types=[
            pltpu.VMEM_SHARED((_ACC_ROWS, _D), jnp.float32),
            pltpu.VMEM((_CH, 128), jnp.int32),
            pltpu.VMEM((_CH, 128), jnp.int32),
            pltpu.VMEM((_CH, 128), jnp.int32),
            pltpu.VMEM((128, _D), jnp.float32),
            pltpu.VMEM((128, _D), jnp.float32),
            pltpu.VMEM((16, _D), jnp.float32),
            pltpu.SemaphoreType.DMA,
            pltpu.SemaphoreType.DMA,
        ],
        compiler_params=pltpu.CompilerParams(use_tc_tiling_on_sc=False),
    )
    return f(yflat, *edge_refs)


def _dinv_body(dp_ref, dinv_ref, dinv2_ref):
    dpa = dp_ref[...]
    deg = dpa[0:4] + dpa[4:8]
    di = jnp.where(deg > 0, 1.0 / jnp.sqrt(jnp.maximum(deg, 1.0)), 0.0)
    dinv_ref[...] = di
    dinv2_ref[...] = di * di


def _dinv_call(deg_part):
    return pl.pallas_call(
        _dinv_body,
        out_shape=(jax.ShapeDtypeStruct((4, _N_PAD), jnp.float32),
                   jax.ShapeDtypeStruct((4, _N_PAD), jnp.float32)),
    )(deg_part.reshape(8, _N_PAD))


_BR = _N_PAD // 8


def _scale1_body(x_ref, s_ref, o_ref):
    b = pl.program_id(1)
    o_ref[0] = x_ref[...] * s_ref[0, 0, pl.ds(b * _BR, _BR)][:, None]


def _scale1_call(emb, dinv):
    return pl.pallas_call(
        _scale1_body,
        grid=(4, 8),
        in_specs=[pl.BlockSpec((_BR, _D), lambda g, b: (b, 0)),
                  pl.BlockSpec((1, 1, _N_PAD), lambda g, b: (g, 0, 0))],
        out_specs=pl.BlockSpec((1, _BR, _D), lambda g, b: (g, b, 0)),
        out_shape=jax.ShapeDtypeStruct((4, _N_PAD, _D), jnp.float32),
    )(emb, dinv.reshape(4, 1, _N_PAD))


def _scale2_body(x_ref, s_ref, o_ref):
    b = pl.program_id(1)
    o_ref[0] = x_ref[0] * s_ref[0, 0, pl.ds(b * _BR, _BR)][:, None]


def _scale2_call(acc, dinv2):
    return pl.pallas_call(
        _scale2_body,
        grid=(4, 8),
        in_specs=[pl.BlockSpec((1, _BR, _D), lambda g, b: (g, b, 0)),
                  pl.BlockSpec((1, 1, _N_PAD), lambda g, b: (g, 0, 0))],
        out_specs=pl.BlockSpec((1, _BR, _D), lambda g, b: (g, b, 0)),
        out_shape=jax.ShapeDtypeStruct((4, _N_PAD, _D), jnp.float32),
    )(acc, dinv2.reshape(4, 1, _N_PAD))


def _loss_body(pu_ref, au_ref, pi_ref, ai_ref, out_ref):
    def cl(p, a):
        pn = p / jnp.maximum(jnp.sqrt(jnp.sum(p * p, axis=1, keepdims=True)), 1e-12)
        an = a / jnp.maximum(jnp.sqrt(jnp.sum(a * a, axis=1, keepdims=True)), 1e-12)
        pos = jnp.exp(jnp.sum(pn * an, axis=1) / _TEMP_S)
        scores = lax.dot_general(pn, an, (((1,), (1,)), ((), ())),
                                 preferred_element_type=jnp.float32)
        ttl = jnp.sum(jnp.exp(scores / _TEMP_S), axis=1)
        return -jnp.mean(jnp.log(pos / ttl))

    c = (cl(pu_ref[...], au_ref[...]) + cl(pi_ref[...], ai_ref[...])) / 2.0
    out_ref[0, 0] = _CON_S * c


def _loss_call(pu, au, pi, ai):
    out = pl.pallas_call(
        _loss_body,
        out_shape=jax.ShapeDtypeStruct((1, 1), jnp.float32),
        out_specs=pl.BlockSpec(memory_space=pltpu.SMEM),
    )(pu, au, pi, ai)
    return out[0, 0]


def kernel(user_emb_glo, item_emb_glo, user_emb_loc, item_emb_loc,
           edges_aux1, edges_aux2, edges_tar, edges_all, edges_aug, batch_data):
    del user_emb_glo, item_emb_glo, edges_all, batch_data
    emb = jnp.concatenate(
        [user_emb_loc, item_emb_loc,
         jnp.zeros((_N_PAD - _N_REAL, _D), jnp.float32)], axis=0)

    graphs = (edges_aux1, edges_aux2, edges_tar, edges_aug)
    rpads = (12544, 12544, 12544, 9472)
    s_degs, s_gats, d_scs = [], [], []
    for g, (e, rp) in enumerate(zip(graphs, rpads)):
        sd, sg, dc = _prep_edges(e, rp, g)
        s_degs.append(sd)
        s_gats.append(sg)
        d_scs.append(dc)

    deg_part = _deg_call(s_degs)
    dinv, dinv2 = _dinv_call(deg_part)

    edge_refs = []
    for sg, dc in zip(s_gats, d_scs):
        edge_refs.extend((sg, dc))

    y1 = _scale1_call(emb, dinv).reshape(4 * _N_PAD, _D)
    acc1 = _prop_call(y1, edge_refs)
    y2 = _scale2_call(acc1, dinv2).reshape(4 * _N_PAD, _D)
    acc2 = _prop_call(y2, edge_refs)

    idx_u = jax.random.permutation(jax.random.key(1), _N_USERS + 1)[:_NB]
    idx_i = jax.random.permutation(jax.random.key(2), _N_USERS + 1)[:_NB]
    sel_u = idx_u.astype(jnp.int32)
    sel_i = idx_i.astype(jnp.int32) + (_N_USERS + 1)

    def side(sel):
        e = emb[sel]
        dv = dinv[:, sel]
        a1 = acc1[:, sel, :] * dv[:, :, None]
        a2 = acc2[:, sel, :] * dv[:, :, None]
        b = (e[None] + a1 + a2) / 3.0
        loc = (b[0] + b[1] + b[2]) / 3.0
        return loc, b[3]

    user_loc_sel, u_aug_sel = side(sel_u)
    item_loc_sel, i_aug_sel = side(sel_i)
    return _loss_call(user_loc_sel, u_aug_sel, item_loc_sel, i_aug_sel)


# speedup vs baseline: 39.6451x; 2.0925x over previous
import jax
import jax.numpy as jnp
from jax import lax
from jax.experimental import pallas as pl
from jax.experimental.pallas import tpu as pltpu
from jax.experimental.pallas import tpu_sc as plsc

_N_USERS = 25000
_D = 64
_TEMP_S = 0.2
_CON_S = 0.1
_NB = 1024

_NU = _N_USERS + 1
_N_PAD = 50176
_HALF = _N_PAD // 2
_TRASH = 256
_ACC_ROWS = _HALF + _TRASH
_PAD_SRC = _N_PAD - 8
_PAD_DST = _HALF + 64
_CH = 8
_NDEPTH = 3


def _prep_edges(edges, rh, g):
    u = edges[0].astype(jnp.int32)
    i = edges[1].astype(jnp.int32)
    pad = rh * 128 - u.shape[0]
    g_n = g * _N_PAD
    padsrc = jnp.full((pad,), g_n + _PAD_SRC, jnp.int32)
    paddst = jnp.full((pad,), _PAD_DST, jnp.int32)
    s = jnp.concatenate([i + (_HALF + g_n), padsrc, u + g_n, padsrc])
    dloc = jnp.concatenate([u, paddst, i, paddst])
    return s.reshape(2 * rh, 128), dloc.reshape(2 * rh, 128)


def _deg_body(s0, s1, s2, s3, out, degs, sbuf, ones, zbuf):
    c = lax.axis_index("c")
    sid = lax.axis_index("s")
    for k in range(8):
        ones[pl.ds(k * 16, 16)] = jnp.ones((16,), jnp.float32)
    for k in range(64):
        zbuf[pl.ds(k * 16, 16)] = jnp.zeros((16,), jnp.float32)
    my = 4 * _N_PAD // 16
    base = sid * my
    for k in range(12):
        pltpu.sync_copy(zbuf, degs.at[pl.ds(base + k * 1024, 1024)])
    pltpu.sync_copy(zbuf.at[pl.ds(0, 256)], degs.at[pl.ds(base + 12288, 256)])
    plsc.subcore_barrier()

    for sref in (s0, s1, s2, s3):
        rh = sref.shape[0] // 2
        per_w = rh // 16
        r0 = c * rh + sid * per_w

        def chunk(cc, carry):
            j0 = r0 + cc * _CH
            pltpu.sync_copy(sref.at[pl.ds(j0, _CH)], sbuf)
            for j in range(_CH):
                pltpu.sync_copy(ones, degs.at[sbuf.at[j]], add=True)
            return carry

        lax.fori_loop(0, per_w // _CH, chunk, 0)

    plsc.subcore_barrier()
    p = c * (4 * _N_PAD)
    pltpu.sync_copy(degs.at[pl.ds(base, my)], out.at[pl.ds(p + base, my)])


def _deg_call(srefs):
    mesh = plsc.VectorSubcoreMesh(core_axis_name="c", subcore_axis_name="s")
    f = pl.kernel(
        _deg_body,
        out_type=jax.ShapeDtypeStruct((2 * 4 * _N_PAD,), jnp.float32),
        mesh=mesh,
        scratch_types=[
            pltpu.VMEM_SHARED((4 * _N_PAD,), jnp.float32),
            pltpu.VMEM((_CH, 128), jnp.int32),
            pltpu.VMEM((128,), jnp.float32),
            pltpu.VMEM((1024,), jnp.float32),
        ],
        compiler_params=pltpu.CompilerParams(use_tc_tiling_on_sc=False),
    )
    return f(*srefs)


def _prop_body(yflat, s0, d0, s1, d1, s2, d2, s3, d3, out, accs, sbuf, dbuf,
               r0b, r1b, r2b, zrows, g0, g1, g2, t0, t1, t2):
    c = lax.axis_index("c")
    sid = lax.axis_index("s")
    lo = c * _HALF
    rows_bufs = (r0b, r1b, r2b)
    gsems = (g0, g1, g2)
    ssems = (t0, t1, t2)
    for r in range(16):
        for k in range(4):
            zrows[r, pl.ds(k * 16, 16)] = jnp.zeros((16,), jnp.float32)
    my_rows = _ACC_ROWS // 16

    for g, (sref, dref) in enumerate(((s0, d0), (s1, d1), (s2, d2), (s3, d3))):
        def zero(z, carry):
            pltpu.sync_copy(zrows, accs.at[pl.ds(sid * my_rows + z * 16, 16)])
            return carry
        lax.fori_loop(0, my_rows // 16, zero, 0)
        plsc.subcore_barrier()

        rh = sref.shape[0] // 2
        per_w = rh // 16
        r0 = c * rh + sid * per_w

        def chunk(cc, carry):
            j0 = r0 + cc * _CH
            pltpu.sync_copy(sref.at[pl.ds(j0, _CH)], sbuf)
            pltpu.sync_copy(dref.at[pl.ds(j0, _CH)], dbuf)
            gds = [None] * _NDEPTH
            sds = [None] * _NDEPTH
            for j in range(_NDEPTH):
                gds[j] = pltpu.async_copy(yflat.at[sbuf.at[j]],
                                          rows_bufs[j], gsems[j])
            for j in range(_CH):
                b = j % _NDEPTH
                gds[b].wait()
                sds[b] = pltpu.async_copy(rows_bufs[b], accs.at[dbuf.at[j]],
                                          ssems[b], add=True)
                if j + _NDEPTH < _CH:
                    sds[b].wait()
                    gds[b] = pltpu.async_copy(yflat.at[sbuf.at[j + _NDEPTH]],
                                              rows_bufs[b], gsems[b])
            for j in range(_CH - _NDEPTH, _CH):
                sds[j % _NDEPTH].wait()
            return carry

        lax.fori_loop(0, per_w // _CH, chunk, 0)
        plsc.subcore_barrier()

        off = sid * (_HALF // 16)
        pltpu.sync_copy(accs.at[pl.ds(off, _HALF // 16)],
                        out.at[g, pl.ds(lo + off, _HALF // 16), :])
        plsc.subcore_barrier()


def _prop_call(yflat, edge_refs):
    mesh = plsc.VectorSubcoreMesh(core_axis_name="c", subcore_axis_name="s")
    f = pl.kernel(
        _prop_body,
        out_type=jax.ShapeDtypeStruct((4, _N_PAD, _D), jnp.float32),
        mesh=mesh,
        scratch_types=(
            [pltpu.VMEM_SHARED((_ACC_ROWS, _D), jnp.float32),
             pltpu.VMEM((_CH, 128), jnp.int32),
             pltpu.VMEM((_CH, 128), jnp.int32)]
            + [pltpu.VMEM((128, _D), jnp.float32) for _ in range(_NDEPTH)]
            + [pltpu.VMEM((16, _D), jnp.float32)]
            + [pltpu.SemaphoreType.DMA for _ in range(2 * _NDEPTH)]
        ),
        compiler_params=pltpu.CompilerParams(use_tc_tiling_on_sc=False),
    )
    return f(yflat, *edge_refs)


def _dinv_body(dp_ref, dinv_ref, dinv2_ref):
    dpa = dp_ref[...]
    deg = dpa[:1568] + dpa[1568:]
    di = jnp.where(deg > 0, 1.0 / jnp.sqrt(jnp.maximum(deg, 1.0)), 0.0)
    dinv_ref[...] = di
    dinv2_ref[...] = di * di


def _dinv_call(deg_part):
    o = pl.pallas_call(
        _dinv_body,
        out_shape=(jax.ShapeDtypeStruct((1568, 128), jnp.float32),
                   jax.ShapeDtypeStruct((1568, 128), jnp.float32)),
    )(deg_part.reshape(3136, 128))
    return o[0].reshape(4, _N_PAD), o[1].reshape(4, _N_PAD)


_BR = _N_PAD // 8


def _scale1_body(x_ref, s_ref, o_ref):
    b = pl.program_id(1)
    o_ref[0] = x_ref[...] * s_ref[0, 0, pl.ds(b * _BR, _BR)][:, None]


def _scale1_call(emb, dinv):
    return pl.pallas_call(
        _scale1_body,
        grid=(4, 8),
        in_specs=[pl.BlockSpec((_BR, _D), lambda g, b: (b, 0)),
                  pl.BlockSpec((1, 1, _N_PAD), lambda g, b: (g, 0, 0))],
        out_specs=pl.BlockSpec((1, _BR, _D), lambda g, b: (g, b, 0)),
        out_shape=jax.ShapeDtypeStruct((4, _N_PAD, _D), jnp.float32),
    )(emb, dinv.reshape(4, 1, _N_PAD))


def _scale2_body(x_ref, s_ref, o_ref):
    b = pl.program_id(1)
    o_ref[0] = x_ref[0] * s_ref[0, 0, pl.ds(b * _BR, _BR)][:, None]


def _scale2_call(acc, dinv2):
    return pl.pallas_call(
        _scale2_body,
        grid=(4, 8),
        in_specs=[pl.BlockSpec((1, _BR, _D), lambda g, b: (g, b, 0)),
                  pl.BlockSpec((1, 1, _N_PAD), lambda g, b: (g, 0, 0))],
        out_specs=pl.BlockSpec((1, _BR, _D), lambda g, b: (g, b, 0)),
        out_shape=jax.ShapeDtypeStruct((4, _N_PAD, _D), jnp.float32),
    )(acc, dinv2.reshape(4, 1, _N_PAD))


def _loss_body(pu_ref, au_ref, pi_ref, ai_ref, out_ref):
    def cl(p, a):
        pn = p / jnp.maximum(jnp.sqrt(jnp.sum(p * p, axis=1, keepdims=True)), 1e-12)
        an = a / jnp.maximum(jnp.sqrt(jnp.sum(a * a, axis=1, keepdims=True)), 1e-12)
        pos = jnp.exp(jnp.sum(pn * an, axis=1) / _TEMP_S)
        scores = lax.dot_general(pn, an, (((1,), (1,)), ((), ())),
                                 preferred_element_type=jnp.float32)
        ttl = jnp.sum(jnp.exp(scores / _TEMP_S), axis=1)
        return -jnp.mean(jnp.log(pos / ttl))

    c = (cl(pu_ref[...], au_ref[...]) + cl(pi_ref[...], ai_ref[...])) / 2.0
    out_ref[0, 0] = _CON_S * c


def _loss_call(pu, au, pi, ai):
    out = pl.pallas_call(
        _loss_body,
        out_shape=jax.ShapeDtypeStruct((1, 1), jnp.float32),
        out_specs=pl.BlockSpec(memory_space=pltpu.SMEM),
    )(pu, au, pi, ai)
    return out[0, 0]


def kernel(user_emb_glo, item_emb_glo, user_emb_loc, item_emb_loc,
           edges_aux1, edges_aux2, edges_tar, edges_all, edges_aug, batch_data):
    del user_emb_glo, item_emb_glo, edges_all, batch_data
    emb = jnp.concatenate(
        [user_emb_loc, jnp.zeros((_HALF - _NU, _D), jnp.float32),
         item_emb_loc, jnp.zeros((_HALF - _NU, _D), jnp.float32)], axis=0)

    graphs = (edges_aux1, edges_aux2, edges_tar, edges_aug)
    rhs = (6272, 6272, 6272, 4736)
    srefs, edge_refs = [], []
    for g, (e, rh) in enumerate(zip(graphs, rhs)):
        s, dloc = _prep_edges(e, rh, g)
        srefs.append(s)
        edge_refs.extend((s, dloc))

    deg_part = _deg_call(srefs)
    dinv, dinv2 = _dinv_call(deg_part)

    y1 = _scale1_call(emb, dinv).reshape(4 * _N_PAD, _D)
    acc1 = _prop_call(y1, edge_refs)
    y2 = _scale2_call(acc1, dinv2).reshape(4 * _N_PAD, _D)
    acc2 = _prop_call(y2, edge_refs)

    idx_u = jax.random.permutation(jax.random.key(1), _NU)[:_NB]
    idx_i = jax.random.permutation(jax.random.key(2), _NU)[:_NB]
    sel_u = idx_u.astype(jnp.int32)
    sel_i = idx_i.astype(jnp.int32) + _HALF

    def side(sel):
        e = emb[sel]
        dv = dinv[:, sel]
        a1 = acc1[:, sel, :] * dv[:, :, None]
        a2 = acc2[:, sel, :] * dv[:, :, None]
        b = (e[None] + a1 + a2) / 3.0
        loc = (b[0] + b[1] + b[2]) / 3.0
        return loc, b[3]

    user_loc_sel, u_aug_sel = side(sel_u)
    item_loc_sel, i_aug_sel = side(sel_i)
    return _loss_call(user_loc_sel, u_aug_sel, item_loc_sel, i_aug_sel)


# speedup vs baseline: 51.4588x; 1.2980x over previous
import jax
import jax.numpy as jnp
from jax import lax
from jax.experimental import pallas as pl
from jax.experimental.pallas import tpu as pltpu
from jax.experimental.pallas import tpu_sc as plsc

_N_USERS = 25000
_D = 64
_TEMP_S = 0.2
_CON_S = 0.1
_NB = 1024

_NU = _N_USERS + 1
_N_PAD = 50176
_HALF = _N_PAD // 2
_TRASH = 256
_ACC_ROWS = _HALF + _TRASH
_PAD_SRC = _N_PAD - 8
_PAD_DST = _HALF + 64
_CH = 8
_NDEPTH = 3


def _prep_edges(edges, rh, g):
    u = edges[0].astype(jnp.int32)
    i = edges[1].astype(jnp.int32)
    pad = rh * 128 - u.shape[0]
    g_n = g * _N_PAD
    padsrc = jnp.full((pad,), g_n + _PAD_SRC, jnp.int32)
    paddst = jnp.full((pad,), _PAD_DST, jnp.int32)
    s = jnp.concatenate([i + (_HALF + g_n), padsrc, u + g_n, padsrc])
    dloc = jnp.concatenate([u, paddst, i, paddst])
    return s.reshape(2 * rh, 128), dloc.reshape(2 * rh, 128)


def _deg_body(s0, s1, s2, s3, out, degs, sbuf, ones, zbuf):
    c = lax.axis_index("c")
    sid = lax.axis_index("s")
    for k in range(8):
        ones[pl.ds(k * 16, 16)] = jnp.ones((16,), jnp.float32)
    for k in range(64):
        zbuf[pl.ds(k * 16, 16)] = jnp.zeros((16,), jnp.float32)
    my = 4 * _N_PAD // 16
    base = sid * my
    for k in range(12):
        pltpu.sync_copy(zbuf, degs.at[pl.ds(base + k * 1024, 1024)])
    pltpu.sync_copy(zbuf.at[pl.ds(0, 256)], degs.at[pl.ds(base + 12288, 256)])
    plsc.subcore_barrier()

    for sref in (s0, s1, s2, s3):
        rh = sref.shape[0] // 2
        per_w = rh // 16
        r0 = c * rh + sid * per_w

        def chunk(cc, carry):
            j0 = r0 + cc * _CH
            pltpu.sync_copy(sref.at[pl.ds(j0, _CH)], sbuf)
            for j in range(_CH):
                pltpu.sync_copy(ones, degs.at[sbuf.at[j]], add=True)
            return carry

        lax.fori_loop(0, per_w // _CH, chunk, 0)

    plsc.subcore_barrier()
    p = c * (4 * _N_PAD)
    pltpu.sync_copy(degs.at[pl.ds(base, my)], out.at[pl.ds(p + base, my)])


def _deg_call(srefs):
    mesh = plsc.VectorSubcoreMesh(core_axis_name="c", subcore_axis_name="s")
    f = pl.kernel(
        _deg_body,
        out_type=jax.ShapeDtypeStruct((2 * 4 * _N_PAD,), jnp.float32),
        mesh=mesh,
        scratch_types=[
            pltpu.VMEM_SHARED((4 * _N_PAD,), jnp.float32),
            pltpu.VMEM((_CH, 128), jnp.int32),
            pltpu.VMEM((128,), jnp.float32),
            pltpu.VMEM((1024,), jnp.float32),
        ],
        compiler_params=pltpu.CompilerParams(use_tc_tiling_on_sc=False),
    )
    return f(*srefs)


def _prop_body(yflat, s0, d0, s1, d1, s2, d2, s3, d3, out, accs, sbuf, dbuf,
               r0b, r1b, r2b, zrows, g0, g1, g2, t0, t1, t2):
    c = lax.axis_index("c")
    sid = lax.axis_index("s")
    lo = c * _HALF
    rows_bufs = (r0b, r1b, r2b)
    gsems = (g0, g1, g2)
    ssems = (t0, t1, t2)
    for r in range(16):
        for k in range(4):
            zrows[r, pl.ds(k * 16, 16)] = jnp.zeros((16,), jnp.float32)
    my_rows = _ACC_ROWS // 16

    for g, (sref, dref) in enumerate(((s0, d0), (s1, d1), (s2, d2), (s3, d3))):
        def zero(z, carry):
            pltpu.sync_copy(zrows, accs.at[pl.ds(sid * my_rows + z * 16, 16)])
            return carry
        lax.fori_loop(0, my_rows // 16, zero, 0)
        plsc.subcore_barrier()

        rh = sref.shape[0] // 2
        per_w = rh // 16
        r0 = c * rh + sid * per_w

        def chunk(cc, carry):
            j0 = r0 + cc * _CH
            pltpu.sync_copy(sref.at[pl.ds(j0, _CH)], sbuf)
            pltpu.sync_copy(dref.at[pl.ds(j0, _CH)], dbuf)
            gds = [None] * _NDEPTH
            sds = [None] * _NDEPTH
            for j in range(_NDEPTH):
                gds[j] = pltpu.async_copy(yflat.at[sbuf.at[j]],
                                          rows_bufs[j], gsems[j])
            for j in range(_CH):
                b = j % _NDEPTH
                gds[b].wait()
                sds[b] = pltpu.async_copy(rows_bufs[b], accs.at[dbuf.at[j]],
                                          ssems[b], add=True)
                if j + _NDEPTH < _CH:
                    sds[b].wait()
                    gds[b] = pltpu.async_copy(yflat.at[sbuf.at[j + _NDEPTH]],
                                              rows_bufs[b], gsems[b])
            for j in range(_CH - _NDEPTH, _CH):
                sds[j % _NDEPTH].wait()
            return carry

        lax.fori_loop(0, per_w // _CH, chunk, 0)
        plsc.subcore_barrier()

        off = sid * (_HALF // 16)
        pltpu.sync_copy(accs.at[pl.ds(off, _HALF // 16)],
                        out.at[g, pl.ds(lo + off, _HALF // 16), :])
        plsc.subcore_barrier()


def _prop_call(yflat, edge_refs):
    mesh = plsc.VectorSubcoreMesh(core_axis_name="c", subcore_axis_name="s")
    f = pl.kernel(
        _prop_body,
        out_type=jax.ShapeDtypeStruct((4, _N_PAD, _D), jnp.float32),
        mesh=mesh,
        scratch_types=(
            [pltpu.VMEM_SHARED((_ACC_ROWS, _D), jnp.float32),
             pltpu.VMEM((_CH, 128), jnp.int32),
             pltpu.VMEM((_CH, 128), jnp.int32)]
            + [pltpu.VMEM((128, _D), jnp.float32) for _ in range(_NDEPTH)]
            + [pltpu.VMEM((16, _D), jnp.float32)]
            + [pltpu.SemaphoreType.DMA for _ in range(2 * _NDEPTH)]
        ),
        compiler_params=pltpu.CompilerParams(use_tc_tiling_on_sc=False),
    )
    return f(yflat, *edge_refs)


_ACC2_ROWS = 1024 + 128


def _prop2_body(yflat, selmap, s0, d0, s1, d1, s2, d2, s3, d3, out,
                accs2, smap_v, sbuf, dbuf, ssurv, dsurv, dfire, rowsb,
                zrows, gsem):
    c = lax.axis_index("c")
    sid = lax.axis_index("s")
    for r in range(8):
        for k in range(4):
            zrows[r, pl.ds(k * 16, 16)] = jnp.zeros((16,), jnp.float32)
    pltpu.sync_copy(selmap.at[c], smap_v)
    my_rows = _ACC2_ROWS // 16

    def fire_batch():
        for t in range(8):
            dfire[0, pl.ds(t * 16, 16)] = dsurv[pl.ds(t * 16, 16)]
        pltpu.async_copy(yflat.at[ssurv.at[pl.ds(0, 128)]], rowsb, gsem).wait()
        pltpu.sync_copy(rowsb, accs2.at[dfire.at[0]], add=True)

    for g, (sref, dref) in enumerate(((s0, d0), (s1, d1), (s2, d2), (s3, d3))):
        def zero(z, carry):
            pltpu.sync_copy(zrows, accs2.at[pl.ds(sid * my_rows + z * 8, 8)])
            return carry
        lax.fori_loop(0, my_rows // 8, zero, 0)
        plsc.subcore_barrier()

        rh = sref.shape[0] // 2
        per_w = rh // 16
        r0 = c * rh + sid * per_w

        def chunk(cc, cs):
            j0 = r0 + cc * _CH
            pltpu.sync_copy(sref.at[pl.ds(j0, _CH)], sbuf)
            pltpu.sync_copy(dref.at[pl.ds(j0, _CH)], dbuf)

            def grp(k, cs):
                j = k // 8
                m = k - j * 8
                sv = sbuf[j, pl.ds(m * 16, 16)]
                dv = dbuf[j, pl.ds(m * 16, 16)]
                slot = plsc.load_gather(smap_v, [dv])
                msk = slot >= 0
                plsc.store_compressed(ssurv.at[pl.ds(cs, 16)], sv, mask=msk)
                plsc.store_compressed(dsurv.at[pl.ds(cs, 16)], slot, mask=msk)
                cs = cs + jnp.sum(msk.astype(jnp.int32))

                @pl.when(cs >= 128)
                def _fire():
                    fire_batch()
                    ssurv[pl.ds(0, 16)] = ssurv[pl.ds(128, 16)]
                    dsurv[pl.ds(0, 16)] = dsurv[pl.ds(128, 16)]

                return jnp.where(cs >= 128, cs - 128, cs)

            return lax.fori_loop(0, _CH * 8, grp, cs)

        cs = lax.fori_loop(0, per_w // _CH, chunk, jnp.int32(0))

        padg = g * _N_PAD + _PAD_SRC
        for t in range(8):
            ssurv[pl.ds(cs + t * 16, 16)] = jnp.full((16,), padg, jnp.int32)
            dsurv[pl.ds(cs + t * 16, 16)] = jnp.full((16,), 1024, jnp.int32)
        fire_batch()
        plsc.subcore_barrier()

        pltpu.sync_copy(accs2.at[pl.ds(sid * 64, 64)],
                        out.at[g, pl.ds(c * 1024 + sid * 64, 64), :])
        plsc.subcore_barrier()


def _prop2_call(yflat, selmap, edge_refs):
    mesh = plsc.VectorSubcoreMesh(core_axis_name="c", subcore_axis_name="s")
    f = pl.kernel(
        _prop2_body,
        out_type=jax.ShapeDtypeStruct((4, 2048, _D), jnp.float32),
        mesh=mesh,
        scratch_types=[
            pltpu.VMEM_SHARED((_ACC2_ROWS, _D), jnp.float32),
            pltpu.VMEM((_ACC_ROWS,), jnp.int32),
            pltpu.VMEM((_CH, 128), jnp.int32),
            pltpu.VMEM((_CH, 128), jnp.int32),
            pltpu.VMEM((512,), jnp.int32),
            pltpu.VMEM((512,), jnp.int32),
            pltpu.VMEM((1, 128), jnp.int32),
            pltpu.VMEM((128, _D), jnp.float32),
            pltpu.VMEM((8, _D), jnp.float32),
            pltpu.SemaphoreType.DMA,
        ],
        compiler_params=pltpu.CompilerParams(use_tc_tiling_on_sc=False,
                                             needs_layout_passes=False),
    )
    return f(yflat, selmap, *edge_refs)


def _dinv_body(dp_ref, dinv_ref, dinv2_ref):
    dpa = dp_ref[...]
    deg = dpa[:1568] + dpa[1568:]
    di = jnp.where(deg > 0, 1.0 / jnp.sqrt(jnp.maximum(deg, 1.0)), 0.0)
    dinv_ref[...] = di
    dinv2_ref[...] = di * di


def _dinv_call(deg_part):
    o = pl.pallas_call(
        _dinv_body,
        out_shape=(jax.ShapeDtypeStruct((1568, 128), jnp.float32),
                   jax.ShapeDtypeStruct((1568, 128), jnp.float32)),
    )(deg_part.reshape(3136, 128))
    return o[0].reshape(4, _N_PAD), o[1].reshape(4, _N_PAD)


_BR = _N_PAD // 8


def _scale1_body(x_ref, s_ref, o_ref):
    b = pl.program_id(1)
    o_ref[0] = x_ref[...] * s_ref[0, 0, pl.ds(b * _BR, _BR)][:, None]


def _scale1_call(emb, dinv):
    return pl.pallas_call(
        _scale1_body,
        grid=(4, 8),
        in_specs=[pl.BlockSpec((_BR, _D), lambda g, b: (b, 0)),
                  pl.BlockSpec((1, 1, _N_PAD), lambda g, b: (g, 0, 0))],
        out_specs=pl.BlockSpec((1, _BR, _D), lambda g, b: (g, b, 0)),
        out_shape=jax.ShapeDtypeStruct((4, _N_PAD, _D), jnp.float32),
    )(emb, dinv.reshape(4, 1, _N_PAD))


def _scale2_body(x_ref, s_ref, o_ref):
    b = pl.program_id(1)
    o_ref[0] = x_ref[0] * s_ref[0, 0, pl.ds(b * _BR, _BR)][:, None]


def _scale2_call(acc, dinv2):
    return pl.pallas_call(
        _scale2_body,
        grid=(4, 8),
        in_specs=[pl.BlockSpec((1, _BR, _D), lambda g, b: (g, b, 0)),
                  pl.BlockSpec((1, 1, _N_PAD), lambda g, b: (g, 0, 0))],
        out_specs=pl.BlockSpec((1, _BR, _D), lambda g, b: (g, b, 0)),
        out_shape=jax.ShapeDtypeStruct((4, _N_PAD, _D), jnp.float32),
    )(acc, dinv2.reshape(4, 1, _N_PAD))


def _loss_body(pu_ref, au_ref, pi_ref, ai_ref, out_ref):
    def cl(p, a):
        pn = p / jnp.maximum(jnp.sqrt(jnp.sum(p * p, axis=1, keepdims=True)), 1e-12)
        an = a / jnp.maximum(jnp.sqrt(jnp.sum(a * a, axis=1, keepdims=True)), 1e-12)
        pos = jnp.exp(jnp.sum(pn * an, axis=1) / _TEMP_S)
        scores = lax.dot_general(pn, an, (((1,), (1,)), ((), ())),
                                 preferred_element_type=jnp.float32)
        ttl = jnp.sum(jnp.exp(scores / _TEMP_S), axis=1)
        return -jnp.mean(jnp.log(pos / ttl))

    c = (cl(pu_ref[...], au_ref[...]) + cl(pi_ref[...], ai_ref[...])) / 2.0
    out_ref[0, 0] = _CON_S * c


def _loss_call(pu, au, pi, ai):
    out = pl.pallas_call(
        _loss_body,
        out_shape=jax.ShapeDtypeStruct((1, 1), jnp.float32),
        out_specs=pl.BlockSpec(memory_space=pltpu.SMEM),
    )(pu, au, pi, ai)
    return out[0, 0]


def kernel(user_emb_glo, item_emb_glo, user_emb_loc, item_emb_loc,
           edges_aux1, edges_aux2, edges_tar, edges_all, edges_aug, batch_data):
    del user_emb_glo, item_emb_glo, edges_all, batch_data
    emb = jnp.concatenate(
        [user_emb_loc, jnp.zeros((_HALF - _NU, _D), jnp.float32),
         item_emb_loc, jnp.zeros((_HALF - _NU, _D), jnp.float32)], axis=0)

    graphs = (edges_aux1, edges_aux2, edges_tar, edges_aug)
    rhs = (6272, 6272, 6272, 4736)
    srefs, edge_refs = [], []
    for g, (e, rh) in enumerate(zip(graphs, rhs)):
        s, dloc = _prep_edges(e, rh, g)
        srefs.append(s)
        edge_refs.extend((s, dloc))

    deg_part = _deg_call(srefs)
    dinv, dinv2 = _dinv_call(deg_part)

    idx_u = jax.random.permutation(jax.random.key(1), _NU)[:_NB]
    idx_i = jax.random.permutation(jax.random.key(2), _NU)[:_NB]
    sel_u = idx_u.astype(jnp.int32)
    sel_i = idx_i.astype(jnp.int32) + _HALF
    slots = jnp.arange(_NB, dtype=jnp.int32)
    selmap = (jnp.full((2, _ACC_ROWS), -1, jnp.int32)
              .at[0, sel_u].set(slots)
              .at[1, sel_i - _HALF].set(slots))

    y1 = _scale1_call(emb, dinv).reshape(4 * _N_PAD, _D)
    acc1 = _prop_call(y1, edge_refs)
    y2 = _scale2_call(acc1, dinv2).reshape(4 * _N_PAD, _D)
    acc2sel = _prop2_call(y2, selmap, edge_refs)

    def side(sel, a2):
        e = emb[sel]
        dv = dinv[:, sel]
        a1 = acc1[:, sel, :] * dv[:, :, None]
        a2 = a2 * dv[:, :, None]
        b = (e[None] + a1 + a2) / 3.0
        loc = (b[0] + b[1] + b[2]) / 3.0
        return loc, b[3]

    user_loc_sel, u_aug_sel = side(sel_u, acc2sel[:, 0:_NB, :])
    item_loc_sel, i_aug_sel = side(sel_i, acc2sel[:, _NB:2 * _NB, :])
    return _loss_call(user_loc_sel, u_aug_sel, item_loc_sel, i_aug_sel)


# speedup vs baseline: 54.9404x; 1.0677x over previous
import jax
import jax.numpy as jnp
from jax import lax
from jax.experimental import pallas as pl
from jax.experimental.pallas import tpu as pltpu
from jax.experimental.pallas import tpu_sc as plsc

_N_USERS = 25000
_D = 64
_TEMP_S = 0.2
_CON_S = 0.1
_NB = 1024

_NU = _N_USERS + 1
_N_PAD = 50176
_HALF = _N_PAD // 2
_TRASH = 256
_ACC_ROWS = _HALF + _TRASH
_PAD_SRC = _N_PAD - 8
_PAD_DST = _HALF + 64
_CH = 8
_NDEPTH = 3


def _prep_edges(edges, rh, g):
    u = edges[0].astype(jnp.int32)
    i = edges[1].astype(jnp.int32)
    pad = rh * 128 - u.shape[0]
    g_n = g * _N_PAD
    padsrc = jnp.full((pad,), g_n + _PAD_SRC, jnp.int32)
    paddst = jnp.full((pad,), _PAD_DST, jnp.int32)
    s = jnp.concatenate([i + (_HALF + g_n), padsrc, u + g_n, padsrc])
    dloc = jnp.concatenate([u, paddst, i, paddst])
    return s.reshape(2 * rh, 128), dloc.reshape(2 * rh, 128)


def _deg_body(s0, s1, s2, s3, out, degs, sbuf, ones, zbuf):
    c = lax.axis_index("c")
    sid = lax.axis_index("s")
    for k in range(8):
        ones[pl.ds(k * 16, 16)] = jnp.ones((16,), jnp.float32)
    for k in range(64):
        zbuf[pl.ds(k * 16, 16)] = jnp.zeros((16,), jnp.float32)
    my = 4 * _N_PAD // 16
    base = sid * my
    for k in range(12):
        pltpu.sync_copy(zbuf, degs.at[pl.ds(base + k * 1024, 1024)])
    pltpu.sync_copy(zbuf.at[pl.ds(0, 256)], degs.at[pl.ds(base + 12288, 256)])
    plsc.subcore_barrier()

    for sref in (s0, s1, s2, s3):
        rh = sref.shape[0] // 2
        per_w = rh // 16
        r0 = c * rh + sid * per_w

        def chunk(cc, carry):
            j0 = r0 + cc * _CH
            pltpu.sync_copy(sref.at[pl.ds(j0, _CH)], sbuf)
            for j in range(_CH):
                pltpu.sync_copy(ones, degs.at[sbuf.at[j]], add=True)
            return carry

        lax.fori_loop(0, per_w // _CH, chunk, 0)

    plsc.subcore_barrier()
    p = c * (4 * _N_PAD)
    pltpu.sync_copy(degs.at[pl.ds(base, my)], out.at[pl.ds(p + base, my)])


def _deg_call(srefs):
    mesh = plsc.VectorSubcoreMesh(core_axis_name="c", subcore_axis_name="s")
    f = pl.kernel(
        _deg_body,
        out_type=jax.ShapeDtypeStruct((2 * 4 * _N_PAD,), jnp.float32),
        mesh=mesh,
        scratch_types=[
            pltpu.VMEM_SHARED((4 * _N_PAD,), jnp.float32),
            pltpu.VMEM((_CH, 128), jnp.int32),
            pltpu.VMEM((128,), jnp.float32),
            pltpu.VMEM((1024,), jnp.float32),
        ],
        compiler_params=pltpu.CompilerParams(use_tc_tiling_on_sc=False),
    )
    return f(*srefs)


def _prop_body(yflat, s0, d0, s1, d1, s2, d2, s3, d3, out, accs, sbuf, dbuf,
               r0b, r1b, r2b, zrows, g0, g1, g2, t0, t1, t2):
    c = lax.axis_index("c")
    sid = lax.axis_index("s")
    lo = c * _HALF
    rows_bufs = (r0b, r1b, r2b)
    gsems = (g0, g1, g2)
    ssems = (t0, t1, t2)
    for r in range(16):
        for k in range(4):
            zrows[r, pl.ds(k * 16, 16)] = jnp.zeros((16,), jnp.float32)
    my_rows = _ACC_ROWS // 16

    for g, (sref, dref) in enumerate(((s0, d0), (s1, d1), (s2, d2), (s3, d3))):
        def zero(z, carry):
            pltpu.sync_copy(zrows, accs.at[pl.ds(sid * my_rows + z * 16, 16)])
            return carry
        lax.fori_loop(0, my_rows // 16, zero, 0)
        plsc.subcore_barrier()

        rh = sref.shape[0] // 2
        per_w = rh // 16
        r0 = c * rh + sid * per_w

        def chunk(cc, carry):
            j0 = r0 + cc * _CH
            pltpu.sync_copy(sref.at[pl.ds(j0, _CH)], sbuf)
            pltpu.sync_copy(dref.at[pl.ds(j0, _CH)], dbuf)
            gds = [None] * _NDEPTH
            sds = [None] * _NDEPTH
            for j in range(_NDEPTH):
                gds[j] = pltpu.async_copy(yflat.at[sbuf.at[j]],
                                          rows_bufs[j], gsems[j])
            for j in range(_CH):
                b = j % _NDEPTH
                gds[b].wait()
                sds[b] = pltpu.async_copy(rows_bufs[b], accs.at[dbuf.at[j]],
                                          ssems[b], add=True)
                if j + _NDEPTH < _CH:
                    sds[b].wait()
                    gds[b] = pltpu.async_copy(yflat.at[sbuf.at[j + _NDEPTH]],
                                              rows_bufs[b], gsems[b])
            for j in range(_CH - _NDEPTH, _CH):
                sds[j % _NDEPTH].wait()
            return carry

        lax.fori_loop(0, per_w // _CH, chunk, 0)
        plsc.subcore_barrier()

        off = sid * (_HALF // 16)
        pltpu.sync_copy(accs.at[pl.ds(off, _HALF // 16)],
                        out.at[g, pl.ds(lo + off, _HALF // 16), :])
        plsc.subcore_barrier()


def _prop_call(yflat, edge_refs):
    mesh = plsc.VectorSubcoreMesh(core_axis_name="c", subcore_axis_name="s")
    f = pl.kernel(
        _prop_body,
        out_type=jax.ShapeDtypeStruct((4, _N_PAD, _D), jnp.float32),
        mesh=mesh,
        scratch_types=(
            [pltpu.VMEM_SHARED((_ACC_ROWS, _D), jnp.float32),
             pltpu.VMEM((_CH, 128), jnp.int32),
             pltpu.VMEM((_CH, 128), jnp.int32)]
            + [pltpu.VMEM((128, _D), jnp.float32) for _ in range(_NDEPTH)]
            + [pltpu.VMEM((16, _D), jnp.float32)]
            + [pltpu.SemaphoreType.DMA for _ in range(2 * _NDEPTH)]
        ),
        compiler_params=pltpu.CompilerParams(use_tc_tiling_on_sc=False),
    )
    return f(yflat, *edge_refs)


_ACC2_ROWS = 1024 + 128


def _prop2_body(yflat, selmap, s0, d0, s1, d1, s2, d2, s3, d3, out,
                accs2, smap_v, sbufa, dbufa, sbufb, dbufb, ssurv, dsurv,
                dfire, rowsb, zrows, gsem, pfa, pfb):
    c = lax.axis_index("c")
    sid = lax.axis_index("s")
    for r in range(8):
        for k in range(4):
            zrows[r, pl.ds(k * 16, 16)] = jnp.zeros((16,), jnp.float32)
    pltpu.sync_copy(selmap.at[c], smap_v)
    my_rows = _ACC2_ROWS // 16

    def fire_batch():
        for t in range(8):
            dfire[0, pl.ds(t * 16, 16)] = dsurv[pl.ds(t * 16, 16)]
        pltpu.async_copy(yflat.at[ssurv.at[pl.ds(0, 128)]], rowsb, gsem).wait()
        pltpu.sync_copy(rowsb, accs2.at[dfire.at[0]], add=True)

    for g, (sref, dref) in enumerate(((s0, d0), (s1, d1), (s2, d2), (s3, d3))):
        def zero(z, carry):
            pltpu.sync_copy(zrows, accs2.at[pl.ds(sid * my_rows + z * 8, 8)])
            return carry
        lax.fori_loop(0, my_rows // 8, zero, 0)
        plsc.subcore_barrier()

        rh = sref.shape[0] // 2
        per_w = rh // 16
        r0 = c * rh + sid * per_w
        nch = per_w // _CH

        def prefetch(cc, sbuf, dbuf, sem):
            j0 = r0 + cc * _CH
            da = pltpu.async_copy(sref.at[pl.ds(j0, _CH)], sbuf, sem)
            db = pltpu.async_copy(dref.at[pl.ds(j0, _CH)], dbuf, sem)
            return da, db

        def walk(sbuf, dbuf, cs):
            def grp(k, cs):
                j = k // 8
                m = k - j * 8
                sv = sbuf[j, pl.ds(m * 16, 16)]
                dv = dbuf[j, pl.ds(m * 16, 16)]
                slot = plsc.load_gather(smap_v, [dv])
                msk = slot >= 0
                plsc.store_compressed(ssurv.at[pl.ds(cs, 16)], sv, mask=msk)
                plsc.store_compressed(dsurv.at[pl.ds(cs, 16)], slot, mask=msk)
                cs = cs + plsc.all_reduce_population_count(msk)[0]

                @pl.when(cs >= 128)
                def _fire():
                    fire_batch()
                    ssurv[pl.ds(0, 16)] = ssurv[pl.ds(128, 16)]
                    dsurv[pl.ds(0, 16)] = dsurv[pl.ds(128, 16)]

                return jnp.where(cs >= 128, cs - 128, cs)

            return lax.fori_loop(0, _CH * 8, grp, cs)

        da, db = prefetch(0, sbufa, dbufa, pfa)
        da.wait()
        db.wait()

        def pair(t, cs):
            cc = 2 * t
            na, nb = prefetch(cc + 1, sbufb, dbufb, pfb)
            cs = walk(sbufa, dbufa, cs)
            na.wait()
            nb.wait()
            nxt = jnp.minimum(cc + 2, nch - 1)
            na2, nb2 = prefetch(nxt, sbufa, dbufa, pfa)
            cs = walk(sbufb, dbufb, cs)
            na2.wait()
            nb2.wait()
            return cs

        cs = lax.fori_loop(0, (nch - 1) // 2, pair, jnp.int32(0))
        cs = walk(sbufa, dbufa, cs)

        padg = g * _N_PAD + _PAD_SRC
        for t in range(8):
            ssurv[pl.ds(cs + t * 16, 16)] = jnp.full((16,), padg, jnp.int32)
            dsurv[pl.ds(cs + t * 16, 16)] = jnp.full((16,), 1024, jnp.int32)
        fire_batch()
        plsc.subcore_barrier()

        pltpu.sync_copy(accs2.at[pl.ds(sid * 64, 64)],
                        out.at[g, pl.ds(c * 1024 + sid * 64, 64), :])
        plsc.subcore_barrier()


def _prop2_call(yflat, selmap, edge_refs):
    mesh = plsc.VectorSubcoreMesh(core_axis_name="c", subcore_axis_name="s")
    f = pl.kernel(
        _prop2_body,
        out_type=jax.ShapeDtypeStruct((4, 2048, _D), jnp.float32),
        mesh=mesh,
        scratch_types=[
            pltpu.VMEM_SHARED((_ACC2_ROWS, _D), jnp.float32),
            pltpu.VMEM((_ACC_ROWS,), jnp.int32),
            pltpu.VMEM((_CH, 128), jnp.int32),
            pltpu.VMEM((_CH, 128), jnp.int32),
            pltpu.VMEM((_CH, 128), jnp.int32),
            pltpu.VMEM((_CH, 128), jnp.int32),
            pltpu.VMEM((512,), jnp.int32),
            pltpu.VMEM((512,), jnp.int32),
            pltpu.VMEM((1, 128), jnp.int32),
            pltpu.VMEM((128, _D), jnp.float32),
            pltpu.VMEM((8, _D), jnp.float32),
            pltpu.SemaphoreType.DMA,
            pltpu.SemaphoreType.DMA,
            pltpu.SemaphoreType.DMA,
        ],
        compiler_params=pltpu.CompilerParams(use_tc_tiling_on_sc=False,
                                             needs_layout_passes=False),
    )
    return f(yflat, selmap, *edge_refs)


def _dinv_body(dp_ref, dinv_ref, dinv2_ref):
    dpa = dp_ref[...]
    deg = dpa[:1568] + dpa[1568:]
    di = jnp.where(deg > 0, 1.0 / jnp.sqrt(jnp.maximum(deg, 1.0)), 0.0)
    dinv_ref[...] = di
    dinv2_ref[...] = di * di


def _dinv_call(deg_part):
    o = pl.pallas_call(
        _dinv_body,
        out_shape=(jax.ShapeDtypeStruct((1568, 128), jnp.float32),
                   jax.ShapeDtypeStruct((1568, 128), jnp.float32)),
    )(deg_part.reshape(3136, 128))
    return o[0].reshape(4, _N_PAD), o[1].reshape(4, _N_PAD)


_BR = _N_PAD // 8


def _scale1_body(x_ref, s_ref, o_ref):
    b = pl.program_id(1)
    o_ref[0] = x_ref[...] * s_ref[0, 0, pl.ds(b * _BR, _BR)][:, None]


def _scale1_call(emb, dinv):
    return pl.pallas_call(
        _scale1_body,
        grid=(4, 8),
        in_specs=[pl.BlockSpec((_BR, _D), lambda g, b: (b, 0)),
                  pl.BlockSpec((1, 1, _N_PAD), lambda g, b: (g, 0, 0))],
        out_specs=pl.BlockSpec((1, _BR, _D), lambda g, b: (g, b, 0)),
        out_shape=jax.ShapeDtypeStruct((4, _N_PAD, _D), jnp.float32),
    )(emb, dinv.reshape(4, 1, _N_PAD))


def _scale2_body(x_ref, s_ref, o_ref):
    b = pl.program_id(1)
    o_ref[0] = x_ref[0] * s_ref[0, 0, pl.ds(b * _BR, _BR)][:, None]


def _scale2_call(acc, dinv2):
    return pl.pallas_call(
        _scale2_body,
        grid=(4, 8),
        in_specs=[pl.BlockSpec((1, _BR, _D), lambda g, b: (g, b, 0)),
                  pl.BlockSpec((1, 1, _N_PAD), lambda g, b: (g, 0, 0))],
        out_specs=pl.BlockSpec((1, _BR, _D), lambda g, b: (g, b, 0)),
        out_shape=jax.ShapeDtypeStruct((4, _N_PAD, _D), jnp.float32),
    )(acc, dinv2.reshape(4, 1, _N_PAD))


def _loss_body(pu_ref, au_ref, pi_ref, ai_ref, out_ref):
    def cl(p, a):
        pn = p / jnp.maximum(jnp.sqrt(jnp.sum(p * p, axis=1, keepdims=True)), 1e-12)
        an = a / jnp.maximum(jnp.sqrt(jnp.sum(a * a, axis=1, keepdims=True)), 1e-12)
        pos = jnp.exp(jnp.sum(pn * an, axis=1) / _TEMP_S)
        scores = lax.dot_general(pn, an, (((1,), (1,)), ((), ())),
                                 preferred_element_type=jnp.float32)
        ttl = jnp.sum(jnp.exp(scores / _TEMP_S), axis=1)
        return -jnp.mean(jnp.log(pos / ttl))

    c = (cl(pu_ref[...], au_ref[...]) + cl(pi_ref[...], ai_ref[...])) / 2.0
    out_ref[0, 0] = _CON_S * c


def _loss_call(pu, au, pi, ai):
    out = pl.pallas_call(
        _loss_body,
        out_shape=jax.ShapeDtypeStruct((1, 1), jnp.float32),
        out_specs=pl.BlockSpec(memory_space=pltpu.SMEM),
    )(pu, au, pi, ai)
    return out[0, 0]


def kernel(user_emb_glo, item_emb_glo, user_emb_loc, item_emb_loc,
           edges_aux1, edges_aux2, edges_tar, edges_all, edges_aug, batch_data):
    del user_emb_glo, item_emb_glo, edges_all, batch_data
    emb = jnp.concatenate(
        [user_emb_loc, jnp.zeros((_HALF - _NU, _D), jnp.float32),
         item_emb_loc, jnp.zeros((_HALF - _NU, _D), jnp.float32)], axis=0)

    graphs = (edges_aux1, edges_aux2, edges_tar, edges_aug)
    rhs = (6272, 6272, 6272, 4736)
    srefs, edge_refs = [], []
    for g, (e, rh) in enumerate(zip(graphs, rhs)):
        s, dloc = _prep_edges(e, rh, g)
        srefs.append(s)
        edge_refs.extend((s, dloc))

    deg_part = _deg_call(srefs)
    dinv, dinv2 = _dinv_call(deg_part)

    idx_u = jax.random.permutation(jax.random.key(1), _NU)[:_NB]
    idx_i = jax.random.permutation(jax.random.key(2), _NU)[:_NB]
    sel_u = idx_u.astype(jnp.int32)
    sel_i = idx_i.astype(jnp.int32) + _HALF
    slots = jnp.arange(_NB, dtype=jnp.int32)
    selmap = (jnp.full((2, _ACC_ROWS), -1, jnp.int32)
              .at[0, sel_u].set(slots)
              .at[1, sel_i - _HALF].set(slots))

    y1 = _scale1_call(emb, dinv).reshape(4 * _N_PAD, _D)
    acc1 = _prop_call(y1, edge_refs)
    y2 = _scale2_call(acc1, dinv2).reshape(4 * _N_PAD, _D)
    acc2sel = _prop2_call(y2, selmap, edge_refs)

    def side(sel, a2):
        e = emb[sel]
        dv = dinv[:, sel]
        a1 = acc1[:, sel, :] * dv[:, :, None]
        a2 = a2 * dv[:, :, None]
        b = (e[None] + a1 + a2) / 3.0
        loc = (b[0] + b[1] + b[2]) / 3.0
        return loc, b[3]

    user_loc_sel, u_aug_sel = side(sel_u, acc2sel[:, 0:_NB, :])
    item_loc_sel, i_aug_sel = side(sel_i, acc2sel[:, _NB:2 * _NB, :])
    return _loss_call(user_loc_sel, u_aug_sel, item_loc_sel, i_aug_sel)


# speedup vs baseline: 57.9297x; 1.0544x over previous
import jax
import jax.numpy as jnp
from jax import lax
from jax.experimental import pallas as pl
from jax.experimental.pallas import tpu as pltpu
from jax.experimental.pallas import tpu_sc as plsc

_N_USERS = 25000
_D = 64
_TEMP_S = 0.2
_CON_S = 0.1
_NB = 1024

_NU = _N_USERS + 1
_N_PAD = 50176
_HALF = _N_PAD // 2
_TRASH = 256
_ACC_ROWS = _HALF + _TRASH
_PAD_SRC = _N_PAD - 8
_PAD_DST = _HALF + 64
_CH = 8
_NDEPTH = 3


def _prep_edges(edges, rh, g):
    u = edges[0].astype(jnp.int32)
    i = edges[1].astype(jnp.int32)
    pad = rh * 128 - u.shape[0]
    g_n = g * _N_PAD
    padsrc = jnp.full((pad,), g_n + _PAD_SRC, jnp.int32)
    paddst = jnp.full((pad,), _PAD_DST, jnp.int32)
    s = jnp.concatenate([i + (_HALF + g_n), padsrc, u + g_n, padsrc])
    dloc = jnp.concatenate([u, paddst, i, paddst])
    return s.reshape(2 * rh, 128), dloc.reshape(2 * rh, 128)


def _deg_body(s0, s1, s2, s3, out, degs, sbuf, ones, zbuf):
    c = lax.axis_index("c")
    sid = lax.axis_index("s")
    for k in range(8):
        ones[pl.ds(k * 16, 16)] = jnp.ones((16,), jnp.float32)
    for k in range(64):
        zbuf[pl.ds(k * 16, 16)] = jnp.zeros((16,), jnp.float32)
    my = 4 * _N_PAD // 16
    base = sid * my
    for k in range(12):
        pltpu.sync_copy(zbuf, degs.at[pl.ds(base + k * 1024, 1024)])
    pltpu.sync_copy(zbuf.at[pl.ds(0, 256)], degs.at[pl.ds(base + 12288, 256)])
    plsc.subcore_barrier()

    for sref in (s0, s1, s2, s3):
        rh = sref.shape[0] // 2
        per_w = rh // 16
        r0 = c * rh + sid * per_w

        def chunk(cc, carry):
            j0 = r0 + cc * _CH
            pltpu.sync_copy(sref.at[pl.ds(j0, _CH)], sbuf)
            for j in range(_CH):
                pltpu.sync_copy(ones, degs.at[sbuf.at[j]], add=True)
            return carry

        lax.fori_loop(0, per_w // _CH, chunk, 0)

    plsc.subcore_barrier()
    p = c * (4 * _N_PAD)
    pltpu.sync_copy(degs.at[pl.ds(base, my)], out.at[pl.ds(p + base, my)])


def _deg_call(srefs):
    mesh = plsc.VectorSubcoreMesh(core_axis_name="c", subcore_axis_name="s")
    f = pl.kernel(
        _deg_body,
        out_type=jax.ShapeDtypeStruct((2 * 4 * _N_PAD,), jnp.float32),
        mesh=mesh,
        scratch_types=[
            pltpu.VMEM_SHARED((4 * _N_PAD,), jnp.float32),
            pltpu.VMEM((_CH, 128), jnp.int32),
            pltpu.VMEM((128,), jnp.float32),
            pltpu.VMEM((1024,), jnp.float32),
        ],
        compiler_params=pltpu.CompilerParams(use_tc_tiling_on_sc=False),
    )
    return f(*srefs)


def _prop_body(yflat, s0, d0, s1, d1, s2, d2, s3, d3, out, accs,
               sbufa, dbufa, sbufb, dbufb, r0b, r1b, r2b,
               g0, g1, g2, t0, t1, t2, pfa, pfb):
    c = lax.axis_index("c")
    sid = lax.axis_index("s")
    lo = c * _HALF
    rows_bufs = (r0b, r1b, r2b)
    gsems = (g0, g1, g2)
    ssems = (t0, t1, t2)
    my_rows = _ACC_ROWS // 16

    for g, (sref, dref) in enumerate(((s0, d0), (s1, d1), (s2, d2), (s3, d3))):
        for r in range(16):
            for k in range(4):
                r0b[r, pl.ds(k * 16, 16)] = jnp.zeros((16,), jnp.float32)

        def zero(z, carry):
            pltpu.sync_copy(r0b.at[pl.ds(0, 16)],
                            accs.at[pl.ds(sid * my_rows + z * 16, 16)])
            return carry
        lax.fori_loop(0, my_rows // 16, zero, 0)
        plsc.subcore_barrier()

        rh = sref.shape[0] // 2
        per_w = rh // 16
        r0 = c * rh + sid * per_w
        nch = per_w // _CH

        def prefetch(cc, sbuf, dbuf, sem):
            j0 = r0 + cc * _CH
            da = pltpu.async_copy(sref.at[pl.ds(j0, _CH)], sbuf, sem)
            db = pltpu.async_copy(dref.at[pl.ds(j0, _CH)], dbuf, sem)
            return da, db

        def ring(sbuf, dbuf):
            gds = [None] * _NDEPTH
            sds = [None] * _NDEPTH
            for j in range(_NDEPTH):
                gds[j] = pltpu.async_copy(yflat.at[sbuf.at[j]],
                                          rows_bufs[j], gsems[j])
            for j in range(_CH):
                b = j % _NDEPTH
                gds[b].wait()
                sds[b] = pltpu.async_copy(rows_bufs[b], accs.at[dbuf.at[j]],
                                          ssems[b], add=True)
                if j + _NDEPTH < _CH:
                    sds[b].wait()
                    gds[b] = pltpu.async_copy(yflat.at[sbuf.at[j + _NDEPTH]],
                                              rows_bufs[b], gsems[b])
            for j in range(_CH - _NDEPTH, _CH):
                sds[j % _NDEPTH].wait()

        da, db = prefetch(0, sbufa, dbufa, pfa)
        da.wait()
        db.wait()

        def pair(t, carry):
            cc = 2 * t
            na, nb = prefetch(cc + 1, sbufb, dbufb, pfb)
            ring(sbufa, dbufa)
            na.wait()
            nb.wait()
            nxt = jnp.minimum(cc + 2, nch - 1)
            na2, nb2 = prefetch(nxt, sbufa, dbufa, pfa)
            ring(sbufb, dbufb)
            na2.wait()
            nb2.wait()
            return carry

        lax.fori_loop(0, (nch - 1) // 2, pair, 0)
        ring(sbufa, dbufa)
        plsc.subcore_barrier()

        off = sid * (_HALF // 16)
        pltpu.sync_copy(accs.at[pl.ds(off, _HALF // 16)],
                        out.at[g, pl.ds(lo + off, _HALF // 16), :])
        plsc.subcore_barrier()


def _prop_call(yflat, edge_refs):
    mesh = plsc.VectorSubcoreMesh(core_axis_name="c", subcore_axis_name="s")
    f = pl.kernel(
        _prop_body,
        out_type=jax.ShapeDtypeStruct((4, _N_PAD, _D), jnp.float32),
        mesh=mesh,
        scratch_types=(
            [pltpu.VMEM_SHARED((_ACC_ROWS, _D), jnp.float32)]
            + [pltpu.VMEM((_CH, 128), jnp.int32) for _ in range(4)]
            + [pltpu.VMEM((128, _D), jnp.float32) for _ in range(_NDEPTH)]
            + [pltpu.SemaphoreType.DMA for _ in range(2 * _NDEPTH + 2)]
        ),
        compiler_params=pltpu.CompilerParams(use_tc_tiling_on_sc=False),
    )
    return f(yflat, *edge_refs)


_ACC2_ROWS = 1024 + 128


def _prop2_body(yflat, selmap, s0, d0, s1, d1, s2, d2, s3, d3, out,
                accs2, smap_v, sbufa, dbufa, sbufb, dbufb, ssurv, dsurv,
                dfire, rowsb, zrows, gsem, pfa, pfb):
    c = lax.axis_index("c")
    sid = lax.axis_index("s")
    for r in range(8):
        for k in range(4):
            zrows[r, pl.ds(k * 16, 16)] = jnp.zeros((16,), jnp.float32)
    pltpu.sync_copy(selmap.at[c], smap_v)
    my_rows = _ACC2_ROWS // 16

    def fire_batch():
        for t in range(8):
            dfire[0, pl.ds(t * 16, 16)] = dsurv[pl.ds(t * 16, 16)]
        pltpu.async_copy(yflat.at[ssurv.at[pl.ds(0, 128)]], rowsb, gsem).wait()
        pltpu.sync_copy(rowsb, accs2.at[dfire.at[0]], add=True)

    for g, (sref, dref) in enumerate(((s0, d0), (s1, d1), (s2, d2), (s3, d3))):
        def zero(z, carry):
            pltpu.sync_copy(zrows, accs2.at[pl.ds(sid * my_rows + z * 8, 8)])
            return carry
        lax.fori_loop(0, my_rows // 8, zero, 0)
        plsc.subcore_barrier()

        rh = sref.shape[0] // 2
        per_w = rh // 16
        r0 = c * rh + sid * per_w
        nch = per_w // _CH

        def prefetch(cc, sbuf, dbuf, sem):
            j0 = r0 + cc * _CH
            da = pltpu.async_copy(sref.at[pl.ds(j0, _CH)], sbuf, sem)
            db = pltpu.async_copy(dref.at[pl.ds(j0, _CH)], dbuf, sem)
            return da, db

        def walk(sbuf, dbuf, cs):
            def grp(k, cs):
                j = k // 8
                m = k - j * 8
                sv = sbuf[j, pl.ds(m * 16, 16)]
                dv = dbuf[j, pl.ds(m * 16, 16)]
                slot = plsc.load_gather(smap_v, [dv])
                msk = slot >= 0
                plsc.store_compressed(ssurv.at[pl.ds(cs, 16)], sv, mask=msk)
                plsc.store_compressed(dsurv.at[pl.ds(cs, 16)], slot, mask=msk)
                cs = cs + plsc.all_reduce_population_count(msk)[0]

                @pl.when(cs >= 128)
                def _fire():
                    fire_batch()
                    ssurv[pl.ds(0, 16)] = ssurv[pl.ds(128, 16)]
                    dsurv[pl.ds(0, 16)] = dsurv[pl.ds(128, 16)]

                return jnp.where(cs >= 128, cs - 128, cs)

            return lax.fori_loop(0, _CH * 8, grp, cs)

        da, db = prefetch(0, sbufa, dbufa, pfa)
        da.wait()
        db.wait()

        def pair(t, cs):
            cc = 2 * t
            na, nb = prefetch(cc + 1, sbufb, dbufb, pfb)
            cs = walk(sbufa, dbufa, cs)
            na.wait()
            nb.wait()
            nxt = jnp.minimum(cc + 2, nch - 1)
            na2, nb2 = prefetch(nxt, sbufa, dbufa, pfa)
            cs = walk(sbufb, dbufb, cs)
            na2.wait()
            nb2.wait()
            return cs

        cs = lax.fori_loop(0, (nch - 1) // 2, pair, jnp.int32(0))
        cs = walk(sbufa, dbufa, cs)

        padg = g * _N_PAD + _PAD_SRC
        for t in range(8):
            ssurv[pl.ds(cs + t * 16, 16)] = jnp.full((16,), padg, jnp.int32)
            dsurv[pl.ds(cs + t * 16, 16)] = jnp.full((16,), 1024, jnp.int32)
        fire_batch()
        plsc.subcore_barrier()

        pltpu.sync_copy(accs2.at[pl.ds(sid * 64, 64)],
                        out.at[g, pl.ds(c * 1024 + sid * 64, 64), :])
        plsc.subcore_barrier()


def _prop2_call(yflat, selmap, edge_refs):
    mesh = plsc.VectorSubcoreMesh(core_axis_name="c", subcore_axis_name="s")
    f = pl.kernel(
        _prop2_body,
        out_type=jax.ShapeDtypeStruct((4, 2048, _D), jnp.float32),
        mesh=mesh,
        scratch_types=[
            pltpu.VMEM_SHARED((_ACC2_ROWS, _D), jnp.float32),
            pltpu.VMEM((_ACC_ROWS,), jnp.int32),
            pltpu.VMEM((_CH, 128), jnp.int32),
            pltpu.VMEM((_CH, 128), jnp.int32),
            pltpu.VMEM((_CH, 128), jnp.int32),
            pltpu.VMEM((_CH, 128), jnp.int32),
            pltpu.VMEM((512,), jnp.int32),
            pltpu.VMEM((512,), jnp.int32),
            pltpu.VMEM((1, 128), jnp.int32),
            pltpu.VMEM((128, _D), jnp.float32),
            pltpu.VMEM((8, _D), jnp.float32),
            pltpu.SemaphoreType.DMA,
            pltpu.SemaphoreType.DMA,
            pltpu.SemaphoreType.DMA,
        ],
        compiler_params=pltpu.CompilerParams(use_tc_tiling_on_sc=False,
                                             needs_layout_passes=False),
    )
    return f(yflat, selmap, *edge_refs)


def _dinv_body(dp_ref, dinv_ref, dinv2_ref):
    dpa = dp_ref[...]
    deg = dpa[:1568] + dpa[1568:]
    di = jnp.where(deg > 0, 1.0 / jnp.sqrt(jnp.maximum(deg, 1.0)), 0.0)
    dinv_ref[...] = di
    dinv2_ref[...] = di * di


def _dinv_call(deg_part):
    o = pl.pallas_call(
        _dinv_body,
        out_shape=(jax.ShapeDtypeStruct((1568, 128), jnp.float32),
                   jax.ShapeDtypeStruct((1568, 128), jnp.float32)),
    )(deg_part.reshape(3136, 128))
    return o[0].reshape(4, _N_PAD), o[1].reshape(4, _N_PAD)


_BR = _N_PAD // 8


def _scale1_body(x_ref, s_ref, o_ref):
    b = pl.program_id(1)
    o_ref[0] = x_ref[...] * s_ref[0, 0, pl.ds(b * _BR, _BR)][:, None]


def _scale1_call(emb, dinv):
    return pl.pallas_call(
        _scale1_body,
        grid=(4, 8),
        in_specs=[pl.BlockSpec((_BR, _D), lambda g, b: (b, 0)),
                  pl.BlockSpec((1, 1, _N_PAD), lambda g, b: (g, 0, 0))],
        out_specs=pl.BlockSpec((1, _BR, _D), lambda g, b: (g, b, 0)),
        out_shape=jax.ShapeDtypeStruct((4, _N_PAD, _D), jnp.float32),
    )(emb, dinv.reshape(4, 1, _N_PAD))


def _scale2_body(x_ref, s_ref, o_ref):
    b = pl.program_id(1)
    o_ref[0] = x_ref[0] * s_ref[0, 0, pl.ds(b * _BR, _BR)][:, None]


def _scale2_call(acc, dinv2):
    return pl.pallas_call(
        _scale2_body,
        grid=(4, 8),
        in_specs=[pl.BlockSpec((1, _BR, _D), lambda g, b: (g, b, 0)),
                  pl.BlockSpec((1, 1, _N_PAD), lambda g, b: (g, 0, 0))],
        out_specs=pl.BlockSpec((1, _BR, _D), lambda g, b: (g, b, 0)),
        out_shape=jax.ShapeDtypeStruct((4, _N_PAD, _D), jnp.float32),
    )(acc, dinv2.reshape(4, 1, _N_PAD))


def _loss_body(pu_ref, au_ref, pi_ref, ai_ref, out_ref):
    def cl(p, a):
        pn = p / jnp.maximum(jnp.sqrt(jnp.sum(p * p, axis=1, keepdims=True)), 1e-12)
        an = a / jnp.maximum(jnp.sqrt(jnp.sum(a * a, axis=1, keepdims=True)), 1e-12)
        pos = jnp.exp(jnp.sum(pn * an, axis=1) / _TEMP_S)
        scores = lax.dot_general(pn, an, (((1,), (1,)), ((), ())),
                                 preferred_element_type=jnp.float32)
        ttl = jnp.sum(jnp.exp(scores / _TEMP_S), axis=1)
        return -jnp.mean(jnp.log(pos / ttl))

    c = (cl(pu_ref[...], au_ref[...]) + cl(pi_ref[...], ai_ref[...])) / 2.0
    out_ref[0, 0] = _CON_S * c


def _loss_call(pu, au, pi, ai):
    out = pl.pallas_call(
        _loss_body,
        out_shape=jax.ShapeDtypeStruct((1, 1), jnp.float32),
        out_specs=pl.BlockSpec(memory_space=pltpu.SMEM),
    )(pu, au, pi, ai)
    return out[0, 0]


def kernel(user_emb_glo, item_emb_glo, user_emb_loc, item_emb_loc,
           edges_aux1, edges_aux2, edges_tar, edges_all, edges_aug, batch_data):
    del user_emb_glo, item_emb_glo, edges_all, batch_data
    emb = jnp.concatenate(
        [user_emb_loc, jnp.zeros((_HALF - _NU, _D), jnp.float32),
         item_emb_loc, jnp.zeros((_HALF - _NU, _D), jnp.float32)], axis=0)

    graphs = (edges_aux1, edges_aux2, edges_tar, edges_aug)
    rhs = (6272, 6272, 6272, 4736)
    srefs, edge_refs = [], []
    for g, (e, rh) in enumerate(zip(graphs, rhs)):
        s, dloc = _prep_edges(e, rh, g)
        srefs.append(s)
        edge_refs.extend((s, dloc))

    deg_part = _deg_call(srefs)
    dinv, dinv2 = _dinv_call(deg_part)

    idx_u = jax.random.permutation(jax.random.key(1), _NU)[:_NB]
    idx_i = jax.random.permutation(jax.random.key(2), _NU)[:_NB]
    sel_u = idx_u.astype(jnp.int32)
    sel_i = idx_i.astype(jnp.int32) + _HALF
    slots = jnp.arange(_NB, dtype=jnp.int32)
    selmap = (jnp.full((2, _ACC_ROWS), -1, jnp.int32)
              .at[0, sel_u].set(slots)
              .at[1, sel_i - _HALF].set(slots))

    y1 = _scale1_call(emb, dinv).reshape(4 * _N_PAD, _D)
    acc1 = _prop_call(y1, edge_refs)
    y2 = _scale2_call(acc1, dinv2).reshape(4 * _N_PAD, _D)
    acc2sel = _prop2_call(y2, selmap, edge_refs)

    def side(sel, a2):
        e = emb[sel]
        dv = dinv[:, sel]
        a1 = acc1[:, sel, :] * dv[:, :, None]
        a2 = a2 * dv[:, :, None]
        b = (e[None] + a1 + a2) / 3.0
        loc = (b[0] + b[1] + b[2]) / 3.0
        return loc, b[3]

    user_loc_sel, u_aug_sel = side(sel_u, acc2sel[:, 0:_NB, :])
    item_loc_sel, i_aug_sel = side(sel_i, acc2sel[:, _NB:2 * _NB, :])
    return _loss_call(user_loc_sel, u_aug_sel, item_loc_sel, i_aug_sel)
